# Initial kernel scaffold; baseline (speedup 1.0000x reference)
import jax
import jax.numpy as jnp
from jax.experimental import pallas as pl


def kernel(gt_inds, anchor_gt_inds, gt_bboxes, proposal_list):
    raise NotImplementedError("write your pallas kernel here")


# speedup vs baseline: 90.2461x; 90.2461x over previous
import functools

import jax
import jax.numpy as jnp
from jax.experimental import pallas as pl
from jax.experimental.pallas import tpu as pltpu

NMS_THR = 0.5
EPS = 1e-6
PULL_W = 1.0
PUSH_W = 1.0
LANES = 128
GP = 64


def _nms_loss_kernel(msk_ref, x1_ref, y1_ref, x2_ref, y2_ref, sc_ref, gt_ref,
                     gtb_ref, gtbT_ref, out_ref, gtmat_ref, *, rows, g_real):
    B = msk_ref.shape[0]
    iota_r = jax.lax.broadcasted_iota(jnp.int32, (rows, LANES), 0)
    iota_c = jax.lax.broadcasted_iota(jnp.int32, (rows, LANES), 1)
    iota3 = iota_r * LANES + iota_c
    lane64 = jax.lax.broadcasted_iota(jnp.int32, (1, GP), 1)

    push_tot = jnp.zeros((), jnp.float32)
    pull_tot = jnp.zeros((), jnp.float32)

    for b in range(B):
        msk = msk_ref[b] > 0.5
        x1v = x1_ref[b]
        y1v = y1_ref[b]
        x2v = x2_ref[b]
        y2v = y2_ref[b]
        scv = sc_ref[b]
        gtf = gt_ref[b]
        areav = (jnp.clip(x2v - x1v, 0.0, None) *
                 jnp.clip(y2v - y1v, 0.0, None))
        logsv = jnp.log(scv)

        gtb = gtb_ref[b]
        gtbT = gtbT_ref[b]
        ax1 = gtbT[:, 0:1]
        ay1 = gtbT[:, 1:2]
        ax2 = gtbT[:, 2:3]
        ay2 = gtbT[:, 3:4]
        bx1 = gtb[0:1, :]
        by1 = gtb[1:2, :]
        bx2 = gtb[2:3, :]
        by2 = gtb[3:4, :]
        area_a = jnp.clip(ax2 - ax1, 0.0, None) * jnp.clip(ay2 - ay1, 0.0, None)
        area_b = jnp.clip(bx2 - bx1, 0.0, None) * jnp.clip(by2 - by1, 0.0, None)
        ltx = jnp.maximum(ax1, bx1)
        lty = jnp.maximum(ay1, by1)
        rbx = jnp.minimum(ax2, bx2)
        rby = jnp.minimum(ay2, by2)
        wx = jnp.clip(rbx - ltx, 0.0, None)
        wy = jnp.clip(rby - lty, 0.0, None)
        inter = wx * wy
        union = area_a + area_b - inter
        gt_iou = inter / jnp.maximum(union, 1e-10)

        acc = jnp.zeros((GP, rows, LANES), jnp.float32)
        for m in range(g_real):
            sel = (gtf == float(m)).astype(jnp.float32)[None, :, :]
            acc = acc + gt_iou[:, m][:, None, None] * sel
        gtmat_ref[...] = acc

        neg = jnp.float32(-jnp.inf)
        zero = jnp.zeros((), jnp.float32)
        seen0 = jnp.zeros((1, GP), jnp.float32)
        gcoord0 = jnp.zeros((1, GP), jnp.float32)

        def cond(state):
            return state[0] > 0.0

        def body(state):
            (nact, activef, seen, gbx1, gby1, gbx2, gby2, garea,
             tp, pc, tpu_a, qc) = state
            active = activef > 0.5
            m = jnp.where(active, scv, neg)
            mx = jnp.max(m)
            ii = jnp.max(jnp.where(m == mx, iota3, -1))
            eq = iota3 == ii
            x1i = jnp.sum(jnp.where(eq, x1v, 0.0))
            y1i = jnp.sum(jnp.where(eq, y1v, 0.0))
            x2i = jnp.sum(jnp.where(eq, x2v, 0.0))
            y2i = jnp.sum(jnp.where(eq, y2v, 0.0))
            s_i = jnp.sum(jnp.where(eq, scv, 0.0))
            gf = jnp.sum(jnp.where(eq, gtf, 0.0))
            gi = gf.astype(jnp.int32)
            area_i = (jnp.clip(x2i - x1i, 0.0, None) *
                      jnp.clip(y2i - y1i, 0.0, None))

            gl = lane64 == gi
            has = jnp.max(jnp.where(gl, seen, 0.0)) > 0.5
            mx1 = jnp.sum(jnp.where(gl, gbx1, 0.0))
            my1 = jnp.sum(jnp.where(gl, gby1, 0.0))
            mx2 = jnp.sum(jnp.where(gl, gbx2, 0.0))
            my2 = jnp.sum(jnp.where(gl, gby2, 0.0))
            marea = jnp.sum(jnp.where(gl, garea, 0.0))

            p_ltx = jnp.maximum(mx1, x1i)
            p_lty = jnp.maximum(my1, y1i)
            p_rbx = jnp.minimum(mx2, x2i)
            p_rby = jnp.minimum(my2, y2i)
            p_w = jnp.clip(p_rbx - p_ltx, 0.0, None)
            p_h = jnp.clip(p_rby - p_lty, 0.0, None)
            p_inter = p_w * p_h
            p_union = marea + area_i - p_inter
            miou = p_inter / jnp.maximum(p_union, 1e-10)
            pull_term = (-jnp.log(1.0 - NMS_THR + jnp.maximum(miou, EPS)) *
                         s_i)
            tp = tp + jnp.where(has, pull_term, 0.0)
            pc = pc + jnp.where(has, 1.0, 0.0)

            upd = gl & jnp.logical_not(has)
            seen = jnp.where(gl, 1.0, seen)
            gbx1 = jnp.where(upd, x1i, gbx1)
            gby1 = jnp.where(upd, y1i, gby1)
            gbx2 = jnp.where(upd, x2i, gbx2)
            gby2 = jnp.where(upd, y2i, gby2)
            garea = jnp.where(upd, area_i, garea)

            rem = active & jnp.logical_not(eq)
            r_ltx = jnp.maximum(x1i, x1v)
            r_lty = jnp.maximum(y1i, y1v)
            r_rbx = jnp.minimum(x2i, x2v)
            r_rby = jnp.minimum(y2i, y2v)
            r_w = jnp.clip(r_rbx - r_ltx, 0.0, None)
            r_h = jnp.clip(r_rby - r_lty, 0.0, None)
            r_inter = r_w * r_h
            r_union = area_i + areav - r_inter
            cur = r_inter / jnp.maximum(r_union, 1e-10)

            ov = rem & (cur > NMS_THR)
            gtrow = gtmat_ref[gi]
            pm = ov & (gtf != gf) & (cur > gtrow)
            cntf = jnp.sum(pm.astype(jnp.float32))
            terms = (-jnp.log(1.0 + NMS_THR - cur) - logsv) * scv
            gsum = jnp.sum(jnp.where(pm, terms, 0.0))
            do_push = cntf > 0.0
            tpu_a = tpu_a + jnp.where(do_push,
                                      gsum / jnp.maximum(cntf, 1.0), 0.0)
            qc = qc + jnp.where(do_push, cntf, 0.0)
            activef = jnp.where(rem & jnp.logical_not(ov), 1.0, 0.0)
            nact = jnp.sum(activef)
            return (nact, activef, seen, gbx1, gby1, gbx2, gby2, garea,
                    tp, pc, tpu_a, qc)

        msk0 = jnp.where(msk, 1.0, 0.0)
        init = (jnp.sum(msk0), msk0, seen0, gcoord0, gcoord0, gcoord0,
                gcoord0, gcoord0, zero, zero, zero, zero)
        final = jax.lax.while_loop(cond, body, init)
        tp, pc, tpu_a, qc = final[8], final[9], final[10], final[11]

        valid = jnp.sum(msk_ref[b]) > 1.0
        push_b = jnp.where(valid, tpu_a / (qc + EPS), 0.0)
        pull_b = jnp.where(valid, tp / (pc + EPS), 0.0)
        push_tot = push_tot + push_b
        pull_tot = pull_tot + pull_b

    out_ref[0] = push_tot / B * PUSH_W
    out_ref[1] = pull_tot / B * PULL_W


@jax.jit
def kernel(gt_inds, anchor_gt_inds, gt_bboxes, proposal_list):
    B, N, _ = proposal_list.shape
    G = gt_bboxes.shape[1]
    rows = -(-N // LANES)
    rows = -(-rows // 8) * 8
    Np = rows * LANES
    pad = Np - N

    prop = proposal_list.astype(jnp.float32)
    x1 = jnp.pad(prop[..., 0], ((0, 0), (0, pad)))
    y1 = jnp.pad(prop[..., 1], ((0, 0), (0, pad)))
    x2 = jnp.pad(prop[..., 2], ((0, 0), (0, pad)))
    y2 = jnp.pad(prop[..., 3], ((0, 0), (0, pad)))
    sc = jnp.pad(prop[..., 4], ((0, 0), (0, pad)), constant_values=1.0)
    agi = anchor_gt_inds.astype(jnp.int32)
    gtf = jnp.pad(agi, ((0, 0), (0, pad))).astype(jnp.float32)
    msk = jnp.pad((agi >= 0).astype(jnp.float32), ((0, 0), (0, pad)))

    shape3 = (B, rows, LANES)
    x1 = x1.reshape(shape3)
    y1 = y1.reshape(shape3)
    x2 = x2.reshape(shape3)
    y2 = y2.reshape(shape3)
    sc = sc.reshape(shape3)
    gtf = gtf.reshape(shape3)
    msk = msk.reshape(shape3)

    gb = gt_bboxes.astype(jnp.float32)
    gbT = jnp.pad(gb, ((0, 0), (0, GP - G), (0, 4)))
    gbb = jnp.pad(jnp.swapaxes(gb, 1, 2),
                  ((0, 0), (0, 4), (0, GP - G)))

    out = pl.pallas_call(
        functools.partial(_nms_loss_kernel, rows=rows, g_real=G),
        out_shape=jax.ShapeDtypeStruct((2,), jnp.float32),
        out_specs=pl.BlockSpec(memory_space=pltpu.SMEM),
        scratch_shapes=[pltpu.VMEM((GP, rows, LANES), jnp.float32)],
    )(msk, x1, y1, x2, y2, sc, gtf, gbb, gbT)
    return out


# speedup vs baseline: 206.0240x; 2.2829x over previous
import functools

import jax
import jax.numpy as jnp
from jax.experimental import pallas as pl
from jax.experimental.pallas import tpu as pltpu

NMS_THR = 0.5
EPS = 1e-6
PULL_W = 1.0
PUSH_W = 1.0
LANES = 128
GP = 64


def _nms_loss_kernel(msk_ref, x1_ref, y1_ref, x2_ref, y2_ref, sc_ref, gt_ref,
                     gtb_ref, gtbT_ref, out_ref, gtmat_ref, *, rows, g_real):
    B = msk_ref.shape[0]
    iota_r = jax.lax.broadcasted_iota(jnp.int32, (1, rows, LANES), 1)
    iota_c = jax.lax.broadcasted_iota(jnp.int32, (1, rows, LANES), 2)
    iota3 = iota_r * LANES + iota_c
    lane64 = jax.lax.broadcasted_iota(jnp.int32, (1, 1, GP), 2)

    mskf = msk_ref[...]
    x1v = x1_ref[...]
    y1v = y1_ref[...]
    x2v = x2_ref[...]
    y2v = y2_ref[...]
    scv = sc_ref[...]
    gtf = gt_ref[...]
    areav = jnp.clip(x2v - x1v, 0.0, None) * jnp.clip(y2v - y1v, 0.0, None)
    logsv = jnp.log(scv)

    for b in range(B):
        gtb = gtb_ref[b]
        gtbT = gtbT_ref[b]
        ax1 = gtbT[:, 0:1]
        ay1 = gtbT[:, 1:2]
        ax2 = gtbT[:, 2:3]
        ay2 = gtbT[:, 3:4]
        bx1 = gtb[0:1, :]
        by1 = gtb[1:2, :]
        bx2 = gtb[2:3, :]
        by2 = gtb[3:4, :]
        area_a = jnp.clip(ax2 - ax1, 0.0, None) * jnp.clip(ay2 - ay1, 0.0, None)
        area_b = jnp.clip(bx2 - bx1, 0.0, None) * jnp.clip(by2 - by1, 0.0, None)
        ltx = jnp.maximum(ax1, bx1)
        lty = jnp.maximum(ay1, by1)
        rbx = jnp.minimum(ax2, bx2)
        rby = jnp.minimum(ay2, by2)
        wx = jnp.clip(rbx - ltx, 0.0, None)
        wy = jnp.clip(rby - lty, 0.0, None)
        inter = wx * wy
        union = area_a + area_b - inter
        gt_iou = inter / jnp.maximum(union, 1e-10)

        gtf_b = gtf[b]
        acc = jnp.zeros((GP, rows, LANES), jnp.float32)
        for m in range(g_real):
            sel = (gtf_b == float(m)).astype(jnp.float32)[None, :, :]
            acc = acc + gt_iou[:, m][:, None, None] * sel
        gtmat_ref[b] = acc

    neg = jnp.float32(-jnp.inf)
    shp_b = (B, 1, 1)
    zero_b = jnp.zeros(shp_b, jnp.float32)
    seen0 = jnp.zeros((B, 1, GP), jnp.float32)

    def cond(state):
        return state[0] > 0.0

    def body(state):
        (_, nact3, activef, seen, gbx1, gby1, gbx2, gby2, garea,
         tp, pc, tpu_a, qc) = state
        alive3 = nact3 > 0.0
        active = activef > 0.5
        m = jnp.where(active, scv, neg)
        mx3 = jnp.max(m, axis=(1, 2), keepdims=True)
        ii3 = jnp.max(jnp.where(m == mx3, iota3, -1), axis=(1, 2),
                      keepdims=True)
        eq = iota3 == ii3

        def _ext(v):
            return jnp.sum(jnp.where(eq, v, 0.0), axis=(1, 2), keepdims=True)

        x1i = _ext(x1v)
        y1i = _ext(y1v)
        x2i = _ext(x2v)
        y2i = _ext(y2v)
        s_i = _ext(scv)
        gf = _ext(gtf)
        gi3 = gf.astype(jnp.int32)
        area_i = (jnp.clip(x2i - x1i, 0.0, None) *
                  jnp.clip(y2i - y1i, 0.0, None))

        gl = lane64 == gi3

        def _gext(v):
            return jnp.sum(jnp.where(gl, v, 0.0), axis=2, keepdims=True)

        has = jnp.max(jnp.where(gl, seen, 0.0), axis=2, keepdims=True) > 0.5
        mx1 = _gext(gbx1)
        my1 = _gext(gby1)
        mx2 = _gext(gbx2)
        my2 = _gext(gby2)
        marea = _gext(garea)

        p_ltx = jnp.maximum(mx1, x1i)
        p_lty = jnp.maximum(my1, y1i)
        p_rbx = jnp.minimum(mx2, x2i)
        p_rby = jnp.minimum(my2, y2i)
        p_w = jnp.clip(p_rbx - p_ltx, 0.0, None)
        p_h = jnp.clip(p_rby - p_lty, 0.0, None)
        p_inter = p_w * p_h
        p_union = marea + area_i - p_inter
        miou = p_inter / jnp.maximum(p_union, 1e-10)
        pull_term = -jnp.log(1.0 - NMS_THR + jnp.maximum(miou, EPS)) * s_i
        do_pull = alive3 & has
        tp = tp + jnp.where(do_pull, pull_term, 0.0)
        pc = pc + jnp.where(do_pull, 1.0, 0.0)

        upd = gl & jnp.logical_not(has) & alive3
        seen = jnp.where(gl & alive3, 1.0, seen)
        gbx1 = jnp.where(upd, x1i, gbx1)
        gby1 = jnp.where(upd, y1i, gby1)
        gbx2 = jnp.where(upd, x2i, gbx2)
        gby2 = jnp.where(upd, y2i, gby2)
        garea = jnp.where(upd, area_i, garea)

        rem = active & jnp.logical_not(eq)
        r_ltx = jnp.maximum(x1i, x1v)
        r_lty = jnp.maximum(y1i, y1v)
        r_rbx = jnp.minimum(x2i, x2v)
        r_rby = jnp.minimum(y2i, y2v)
        r_w = jnp.clip(r_rbx - r_ltx, 0.0, None)
        r_h = jnp.clip(r_rby - r_lty, 0.0, None)
        r_inter = r_w * r_h
        r_union = area_i + areav - r_inter
        cur = r_inter / jnp.maximum(r_union, 1e-10)

        ov = rem & (cur > NMS_THR)
        gtrows = [gtmat_ref[b, jnp.sum(gi3[b])] for b in range(B)]
        gtrow = jnp.stack(gtrows, axis=0)
        pm = ov & (gtf != gf) & (cur > gtrow)
        cnt3 = jnp.sum(pm.astype(jnp.float32), axis=(1, 2), keepdims=True)
        terms = (-jnp.log(1.0 + NMS_THR - cur) - logsv) * scv
        gsum3 = jnp.sum(jnp.where(pm, terms, 0.0), axis=(1, 2), keepdims=True)
        do_push = alive3 & (cnt3 > 0.0)
        tpu_a = tpu_a + jnp.where(do_push, gsum3 / jnp.maximum(cnt3, 1.0),
                                  0.0)
        qc = qc + jnp.where(do_push, cnt3, 0.0)
        activef = jnp.where(rem & jnp.logical_not(ov), 1.0, 0.0)
        nact3 = jnp.sum(activef, axis=(1, 2), keepdims=True)
        ntot = jnp.sum(nact3)
        return (ntot, nact3, activef, seen, gbx1, gby1, gbx2, gby2, garea,
                tp, pc, tpu_a, qc)

    msk0 = jnp.where(mskf > 0.5, 1.0, 0.0)
    nact0 = jnp.sum(msk0, axis=(1, 2), keepdims=True)
    zero_g = jnp.zeros((B, 1, GP), jnp.float32)
    init = (jnp.sum(nact0), nact0, msk0, seen0, zero_g, zero_g, zero_g,
            zero_g, zero_g, zero_b, zero_b, zero_b, zero_b)
    final = jax.lax.while_loop(cond, body, init)
    tp, pc, tpu_a, qc = final[9], final[10], final[11], final[12]

    valid3 = nact0 > 1.0
    push3 = jnp.where(valid3, tpu_a / (qc + EPS), 0.0)
    pull3 = jnp.where(valid3, tp / (pc + EPS), 0.0)
    out_ref[0] = jnp.sum(push3) / B * PUSH_W
    out_ref[1] = jnp.sum(pull3) / B * PULL_W


@jax.jit
def kernel(gt_inds, anchor_gt_inds, gt_bboxes, proposal_list):
    B, N, _ = proposal_list.shape
    G = gt_bboxes.shape[1]
    rows = -(-N // LANES)
    rows = -(-rows // 8) * 8
    Np = rows * LANES
    pad = Np - N

    prop = proposal_list.astype(jnp.float32)
    x1 = jnp.pad(prop[..., 0], ((0, 0), (0, pad)))
    y1 = jnp.pad(prop[..., 1], ((0, 0), (0, pad)))
    x2 = jnp.pad(prop[..., 2], ((0, 0), (0, pad)))
    y2 = jnp.pad(prop[..., 3], ((0, 0), (0, pad)))
    sc = jnp.pad(prop[..., 4], ((0, 0), (0, pad)), constant_values=1.0)
    agi = anchor_gt_inds.astype(jnp.int32)
    gtf = jnp.pad(agi, ((0, 0), (0, pad))).astype(jnp.float32)
    msk = jnp.pad((agi >= 0).astype(jnp.float32), ((0, 0), (0, pad)))

    shape3 = (B, rows, LANES)
    x1 = x1.reshape(shape3)
    y1 = y1.reshape(shape3)
    x2 = x2.reshape(shape3)
    y2 = y2.reshape(shape3)
    sc = sc.reshape(shape3)
    gtf = gtf.reshape(shape3)
    msk = msk.reshape(shape3)

    gb = gt_bboxes.astype(jnp.float32)
    gbT = jnp.pad(gb, ((0, 0), (0, GP - G), (0, 4)))
    gbb = jnp.pad(jnp.swapaxes(gb, 1, 2),
                  ((0, 0), (0, 4), (0, GP - G)))

    out = pl.pallas_call(
        functools.partial(_nms_loss_kernel, rows=rows, g_real=G),
        out_shape=jax.ShapeDtypeStruct((2,), jnp.float32),
        out_specs=pl.BlockSpec(memory_space=pltpu.SMEM),
        scratch_shapes=[pltpu.VMEM((B, GP, rows, LANES), jnp.float32)],
    )(msk, x1, y1, x2, y2, sc, gtf, gbb, gbT)
    return out
